# baseline (speedup 1.0000x reference)
import functools

import jax
import jax.numpy as jnp
from jax import lax
from jax.experimental import pallas as pl
from jax.experimental.pallas import tpu as pltpu
from jax.experimental.pallas import tpu_sc as plsc

EMBEDDING_DIM = 64

_NUM_WORKERS = 32
_CHUNK = 128
_NBUF = 4


def _worker_id():
    info = plsc.get_sparse_core_info()
    return lax.axis_index("s") * info.num_cores + lax.axis_index("c")


def _make_gather(total, dim):
    per_w = total // _NUM_WORKERS
    n_chunk = per_w // _CHUNK
    n_group = n_chunk // _NBUF

    mesh = plsc.VectorSubcoreMesh(core_axis_name="c", subcore_axis_name="s")

    @functools.partial(
        pl.kernel,
        mesh=mesh,
        out_type=jax.ShapeDtypeStruct((total, dim), jnp.float32),
        scratch_types=[
            pltpu.VMEM((n_chunk, _CHUNK), jnp.int32),
            pltpu.VMEM((2 * _NBUF, _CHUNK, dim), jnp.float32),
            pltpu.SemaphoreType.DMA,
            pltpu.SemaphoreType.DMA,
        ],
        compiler_params=pltpu.CompilerParams(use_tc_tiling_on_sc=False),
    )
    def gather(table_hbm, idx_hbm, out_hbm, idx_v, rows_v, gsem, psem):
        wid = _worker_id()
        chunk0 = wid * n_chunk
        base = wid * per_w

        pltpu.sync_copy(idx_hbm.at[pl.ds(chunk0, n_chunk)], idx_v)

        def fire_gather(g, p, b):
            j = g * _NBUF + b
            pltpu.make_async_copy(
                table_hbm.at[idx_v.at[j]],
                rows_v.at[p * _NBUF + b],
                gsem,
            ).start()

        def wait_gather(p, b):
            pltpu.make_async_copy(
                table_hbm.at[idx_v.at[0]],
                rows_v.at[p * _NBUF + b],
                gsem,
            ).wait()

        def fire_put(g, p, b):
            j = g * _NBUF + b
            pltpu.make_async_copy(
                rows_v.at[p * _NBUF + b],
                out_hbm.at[pl.ds(base + j * _CHUNK, _CHUNK)],
                psem,
            ).start()

        def wait_put(p, b):
            pltpu.make_async_copy(
                rows_v.at[p * _NBUF + b],
                out_hbm.at[pl.ds(base, _CHUNK)],
                psem,
            ).wait()

        for b in range(_NBUF):
            fire_gather(0, 0, b)

        def body(g, carry):
            p = g % 2
            q = 1 - p
            for b in range(_NBUF):
                wait_gather(p, b)
            @pl.when(g > 0)
            def _():
                for b in range(_NBUF):
                    wait_put(q, b)
            @pl.when(g + 1 < n_group)
            def _():
                for b in range(_NBUF):
                    fire_gather(g + 1, q, b)
            for b in range(_NBUF):
                fire_put(g, p, b)
            return carry

        lax.fori_loop(0, n_group, body, 0)

        last_p = (n_group - 1) % 2
        for b in range(_NBUF):
            wait_put(last_p, b)

    return gather


def kernel(x, weight):
    batch, seq = x.shape
    total = batch * seq
    idx = x.reshape(total // _CHUNK, _CHUNK).astype(jnp.int32)
    out = _make_gather(total, EMBEDDING_DIM)(weight, idx)
    return out.reshape(batch, seq, EMBEDDING_DIM)


# speedup vs baseline: 1.2143x; 1.2143x over previous
import functools

import jax
import jax.numpy as jnp
from jax import lax
from jax.experimental import pallas as pl
from jax.experimental.pallas import tpu as pltpu
from jax.experimental.pallas import tpu_sc as plsc

EMBEDDING_DIM = 64
_PADDED_DIM = 128

_NUM_WORKERS = 32
_CHUNK = 128
_NBUF = 2


def _worker_id():
    info = plsc.get_sparse_core_info()
    return lax.axis_index("s") * info.num_cores + lax.axis_index("c")


def _make_gather(total, dim):
    per_w = total // _NUM_WORKERS
    n_chunk = per_w // _CHUNK
    n_group = n_chunk // _NBUF

    mesh = plsc.VectorSubcoreMesh(core_axis_name="c", subcore_axis_name="s")

    @functools.partial(
        pl.kernel,
        mesh=mesh,
        out_type=jax.ShapeDtypeStruct((total, _PADDED_DIM), jnp.float32),
        scratch_types=[
            pltpu.VMEM((n_chunk, _CHUNK), jnp.int32),
            pltpu.VMEM((2 * _NBUF, _CHUNK, _PADDED_DIM), jnp.float32),
            pltpu.SemaphoreType.DMA,
            pltpu.SemaphoreType.DMA,
        ],
        compiler_params=pltpu.CompilerParams(use_tc_tiling_on_sc=True),
    )
    def gather(table_hbm, idx_hbm, out_hbm, idx_v, rows_v, gsem, psem):
        wid = _worker_id()
        chunk0 = wid * n_chunk
        base = wid * per_w

        pltpu.sync_copy(idx_hbm.at[pl.ds(chunk0, n_chunk)], idx_v)

        def fire_gather(g, p, b):
            j = g * _NBUF + b
            pltpu.make_async_copy(
                table_hbm.at[idx_v.at[j]],
                rows_v.at[p * _NBUF + b],
                gsem,
            ).start()

        def wait_gather(p, b):
            pltpu.make_async_copy(
                table_hbm.at[idx_v.at[0]],
                rows_v.at[p * _NBUF + b],
                gsem,
            ).wait()

        def fire_put(g, p, b):
            j = g * _NBUF + b
            pltpu.make_async_copy(
                rows_v.at[p * _NBUF + b],
                out_hbm.at[pl.ds(base + j * _CHUNK, _CHUNK)],
                psem,
            ).start()

        def wait_put(p, b):
            pltpu.make_async_copy(
                rows_v.at[p * _NBUF + b],
                out_hbm.at[pl.ds(base, _CHUNK)],
                psem,
            ).wait()

        for b in range(_NBUF):
            fire_gather(0, 0, b)

        def body(g, carry):
            p = g % 2
            q = 1 - p
            for b in range(_NBUF):
                wait_gather(p, b)
            @pl.when(g > 0)
            def _():
                for b in range(_NBUF):
                    wait_put(q, b)
            @pl.when(g + 1 < n_group)
            def _():
                for b in range(_NBUF):
                    fire_gather(g + 1, q, b)
            for b in range(_NBUF):
                fire_put(g, p, b)
            return carry

        lax.fori_loop(0, n_group, body, 0)

        last_p = (n_group - 1) % 2
        for b in range(_NBUF):
            wait_put(last_p, b)

    return gather


def kernel(x, weight):
    batch, seq = x.shape
    total = batch * seq
    idx = x.reshape(total // _CHUNK, _CHUNK).astype(jnp.int32)
    wpad = jnp.pad(weight, ((0, 0), (0, _PADDED_DIM - EMBEDDING_DIM)))
    out = _make_gather(total, EMBEDDING_DIM)(wpad, idx)
    return out[:, :EMBEDDING_DIM].reshape(batch, seq, EMBEDDING_DIM)
